# baseline (speedup 1.0000x reference)
import jax
import jax.numpy as jnp
from jax.experimental import pallas as pl
from jax.experimental.pallas import tpu as pltpu

_N = 65536
_DIM = 512
_NC = 2
_K = 655
_R = 512
_C = 128
_TILE = 2048


def _logits_kernel(h_ref, w_ref, b_ref, out_ref):
    acc = jnp.dot(h_ref[...], w_ref[...], preferred_element_type=jnp.float32)
    out_ref[...] = acc + b_ref[0:1, 0:_NC]


def _select_kernel(
    ai_ref,
    ao_ref,
    p0_ref,
    p1_ref,
    lu_ref,
    ls_ref,
    k0_ref,
    k1_ref,
    k2_ref,
    rm0_ref,
    rm1_ref,
    rm2_ref,
):
    riota = (
        jax.lax.broadcasted_iota(jnp.int32, (4, _C), 0) * _C
        + jax.lax.broadcasted_iota(jnp.int32, (4, _C), 1)
    )
    lane = jax.lax.broadcasted_iota(jnp.int32, (1, _C), 1)
    neg_inf = jnp.float32(-jnp.inf)
    big = jnp.int32(_N)

    k0_ref[:, 0:_C] = ai_ref[...]
    k1_ref[:, 0:_C] = -ai_ref[...]
    k2_ref[:, 0:_C] = ao_ref[...]
    for _kref in (k0_ref, k1_ref, k2_ref):
        _kref[:, _C : 2 * _C] = p0_ref[...]
        _kref[:, 2 * _C : 3 * _C] = p1_ref[...]
    rm0_ref[...] = jnp.max(k0_ref[:, 0:_C].reshape(4, _C, _C), axis=2)
    rm1_ref[...] = jnp.max(k1_ref[:, 0:_C].reshape(4, _C, _C), axis=2)
    rm2_ref[...] = jnp.max(k2_ref[:, 0:_C].reshape(4, _C, _C), axis=2)

    def step(k_ref, rm_ref, orow):
        rm = rm_ref[...]
        m = jnp.max(rm)
        r0 = jnp.min(jnp.where(rm == m, riota, big))
        row = k_ref[pl.ds(r0, 1), :]
        krow = row[:, 0:_C]
        c0 = jnp.min(jnp.where(krow == m, lane, big))
        sel = lane == c0
        nrow = jnp.where(sel, neg_inf, krow)
        k_ref[pl.ds(r0, 1), :] = jnp.concatenate([nrow, row[:, _C:]], axis=1)
        rm_ref[...] = jnp.where(riota == r0, jnp.max(nrow), rm)
        p0 = jnp.sum(jnp.where(sel, row[:, _C : 2 * _C], 0.0))
        p1 = jnp.sum(jnp.where(sel, row[:, 2 * _C : 3 * _C], 0.0))
        lu_ref[pl.ds(orow, 1), :] = jnp.concatenate(
            [jnp.reshape(p0, (1, 1)), jnp.reshape(p1, (1, 1))], axis=1
        )

    def body(j, _):
        step(k0_ref, rm0_ref, j)
        step(k1_ref, rm1_ref, j + _K)
        step(k2_ref, rm2_ref, j + 2 * _K)
        return 0

    jax.lax.fori_loop(0, _K, body, 0)

    lu = lu_ref[...]
    mx = jnp.max(lu, axis=1, keepdims=True)
    e = jnp.exp(lu - mx)
    ls_ref[...] = e / jnp.sum(e, axis=1, keepdims=True)


def kernel(bag_label, h, A, W, b):
    bl = jnp.asarray(bag_label)
    a0 = A[:, 0, 0]
    a1 = A[:, 0, 1]
    is_last = bl == (_NC - 1)
    ai = jnp.where(is_last, a1, a0).reshape(_R, _C)
    ao = jnp.where(is_last, a0, a1).reshape(_R, _C)

    b_pad = jnp.zeros((8, 128), jnp.float32).at[0, :_NC].set(b)

    p = pl.pallas_call(
        _logits_kernel,
        grid=(_N // _TILE,),
        in_specs=[
            pl.BlockSpec((_TILE, _DIM), lambda i: (i, 0)),
            pl.BlockSpec((_DIM, _NC), lambda i: (0, 0)),
            pl.BlockSpec((8, 128), lambda i: (0, 0)),
        ],
        out_specs=pl.BlockSpec((_TILE, _NC), lambda i: (i, 0)),
        out_shape=jax.ShapeDtypeStruct((_N, _NC), jnp.float32),
    )(h, W, b_pad)

    p0 = p[:, 0].reshape(_R, _C)
    p1 = p[:, 1].reshape(_R, _C)

    lu, ls = pl.pallas_call(
        _select_kernel,
        in_specs=[
            pl.BlockSpec((_R, _C), lambda: (0, 0)),
            pl.BlockSpec((_R, _C), lambda: (0, 0)),
            pl.BlockSpec((_R, _C), lambda: (0, 0)),
            pl.BlockSpec((_R, _C), lambda: (0, 0)),
        ],
        out_specs=[
            pl.BlockSpec((3 * _K, _NC), lambda: (0, 0)),
            pl.BlockSpec((3 * _K, _NC), lambda: (0, 0)),
        ],
        out_shape=[
            jax.ShapeDtypeStruct((3 * _K, _NC), jnp.float32),
            jax.ShapeDtypeStruct((3 * _K, _NC), jnp.float32),
        ],
        scratch_shapes=[
            pltpu.VMEM((_R, 3 * _C), jnp.float32),
            pltpu.VMEM((_R, 3 * _C), jnp.float32),
            pltpu.VMEM((_R, 3 * _C), jnp.float32),
            pltpu.VMEM((4, _C), jnp.float32),
            pltpu.VMEM((4, _C), jnp.float32),
            pltpu.VMEM((4, _C), jnp.float32),
        ],
    )(ai, ao, p0, p1)

    labels = jnp.concatenate(
        [
            jnp.ones((_K,), dtype=jnp.int32),
            jnp.zeros((2 * _K,), dtype=jnp.int32),
        ]
    )
    return labels, lu, ls


# speedup vs baseline: 1.0746x; 1.0746x over previous
import jax
import jax.numpy as jnp
from jax.experimental import pallas as pl
from jax.experimental.pallas import tpu as pltpu

_N = 65536
_DIM = 512
_NC = 2
_K = 655
_R = 512
_C = 128
_TILE = 2048


def _logits_kernel(h_ref, w_ref, b_ref, out_ref):
    acc = jnp.dot(h_ref[...], w_ref[...], preferred_element_type=jnp.float32)
    out_ref[...] = acc + b_ref[0:1, 0:_NC]


def _select_kernel(
    ai_ref,
    ao_ref,
    p0_ref,
    p1_ref,
    lu_ref,
    ls_ref,
    k0_ref,
    k1_ref,
    k2_ref,
    rm0_ref,
    rm1_ref,
    rm2_ref,
):
    riota = (
        jax.lax.broadcasted_iota(jnp.int32, (4, _C), 0) * _C
        + jax.lax.broadcasted_iota(jnp.int32, (4, _C), 1)
    )
    lane = jax.lax.broadcasted_iota(jnp.int32, (1, _C), 1)
    neg_inf = jnp.float32(-jnp.inf)
    big = jnp.int32(_N)

    k0_ref[:, 0:_C] = ai_ref[...]
    k1_ref[:, 0:_C] = -ai_ref[...]
    k2_ref[:, 0:_C] = ao_ref[...]
    for _kref in (k0_ref, k1_ref, k2_ref):
        _kref[:, _C : 2 * _C] = p0_ref[...]
        _kref[:, 2 * _C : 3 * _C] = p1_ref[...]
    rm0_ref[...] = jnp.max(k0_ref[:, 0:_C].reshape(4, _C, _C), axis=2)
    rm1_ref[...] = jnp.max(k1_ref[:, 0:_C].reshape(4, _C, _C), axis=2)
    rm2_ref[...] = jnp.max(k2_ref[:, 0:_C].reshape(4, _C, _C), axis=2)

    def step(k_ref, rm_ref, orow):
        rm = rm_ref[...]
        mv = jnp.max(rm, keepdims=True)
        r0 = jnp.min(jnp.where(rm == mv, riota, big))
        row = k_ref[pl.ds(r0, 1), :]
        krow = row[:, 0:_C]
        sel0 = krow == mv
        c0v = jnp.min(jnp.where(sel0, lane, big), axis=1, keepdims=True)
        sel = lane == c0v
        nrow = jnp.where(sel, neg_inf, krow)
        k_ref[pl.ds(r0, 1), :] = jnp.concatenate([nrow, row[:, _C:]], axis=1)
        rm_ref[...] = jnp.where(riota == r0, jnp.max(nrow, keepdims=True), rm)
        p0 = jnp.sum(jnp.where(sel, row[:, _C : 2 * _C], 0.0), axis=1, keepdims=True)
        p1 = jnp.sum(
            jnp.where(sel, row[:, 2 * _C : 3 * _C], 0.0), axis=1, keepdims=True
        )
        lu_ref[pl.ds(orow, 1), :] = jnp.concatenate([p0, p1], axis=1)

    def body(j, _):
        step(k0_ref, rm0_ref, j)
        step(k1_ref, rm1_ref, j + _K)
        step(k2_ref, rm2_ref, j + 2 * _K)
        return 0

    jax.lax.fori_loop(0, _K, body, 0)

    lu = lu_ref[...]
    mx = jnp.max(lu, axis=1, keepdims=True)
    e = jnp.exp(lu - mx)
    ls_ref[...] = e / jnp.sum(e, axis=1, keepdims=True)


def kernel(bag_label, h, A, W, b):
    bl = jnp.asarray(bag_label)
    a0 = A[:, 0, 0]
    a1 = A[:, 0, 1]
    is_last = bl == (_NC - 1)
    ai = jnp.where(is_last, a1, a0).reshape(_R, _C)
    ao = jnp.where(is_last, a0, a1).reshape(_R, _C)

    b_pad = jnp.zeros((8, 128), jnp.float32).at[0, :_NC].set(b)

    p = pl.pallas_call(
        _logits_kernel,
        grid=(_N // _TILE,),
        in_specs=[
            pl.BlockSpec((_TILE, _DIM), lambda i: (i, 0)),
            pl.BlockSpec((_DIM, _NC), lambda i: (0, 0)),
            pl.BlockSpec((8, 128), lambda i: (0, 0)),
        ],
        out_specs=pl.BlockSpec((_TILE, _NC), lambda i: (i, 0)),
        out_shape=jax.ShapeDtypeStruct((_N, _NC), jnp.float32),
    )(h, W, b_pad)

    p0 = p[:, 0].reshape(_R, _C)
    p1 = p[:, 1].reshape(_R, _C)

    lu, ls = pl.pallas_call(
        _select_kernel,
        in_specs=[
            pl.BlockSpec((_R, _C), lambda: (0, 0)),
            pl.BlockSpec((_R, _C), lambda: (0, 0)),
            pl.BlockSpec((_R, _C), lambda: (0, 0)),
            pl.BlockSpec((_R, _C), lambda: (0, 0)),
        ],
        out_specs=[
            pl.BlockSpec((3 * _K, _NC), lambda: (0, 0)),
            pl.BlockSpec((3 * _K, _NC), lambda: (0, 0)),
        ],
        out_shape=[
            jax.ShapeDtypeStruct((3 * _K, _NC), jnp.float32),
            jax.ShapeDtypeStruct((3 * _K, _NC), jnp.float32),
        ],
        scratch_shapes=[
            pltpu.VMEM((_R, 3 * _C), jnp.float32),
            pltpu.VMEM((_R, 3 * _C), jnp.float32),
            pltpu.VMEM((_R, 3 * _C), jnp.float32),
            pltpu.VMEM((4, _C), jnp.float32),
            pltpu.VMEM((4, _C), jnp.float32),
            pltpu.VMEM((4, _C), jnp.float32),
        ],
    )(ai, ao, p0, p1)

    labels = jnp.concatenate(
        [
            jnp.ones((_K,), dtype=jnp.int32),
            jnp.zeros((2 * _K,), dtype=jnp.int32),
        ]
    )
    return labels, lu, ls


# speedup vs baseline: 1.1365x; 1.0576x over previous
import jax
import jax.numpy as jnp
from jax.experimental import pallas as pl
from jax.experimental.pallas import tpu as pltpu

_N = 65536
_DIM = 512
_NC = 2
_K = 655
_R = 512
_C = 128
_TILE = 2048


def _logits_kernel(h_ref, w_ref, b_ref, out_ref):
    acc = jnp.dot(h_ref[...], w_ref[...], preferred_element_type=jnp.float32)
    out_ref[...] = acc + b_ref[0:1, 0:_NC]


def _select_kernel(
    ai_ref,
    ao_ref,
    p0_ref,
    p1_ref,
    lu_ref,
    ls_ref,
    k0_ref,
    k1_ref,
    k2_ref,
    rm0_ref,
    rm1_ref,
    rm2_ref,
):
    riota = (
        jax.lax.broadcasted_iota(jnp.int32, (4, _C), 0) * _C
        + jax.lax.broadcasted_iota(jnp.int32, (4, _C), 1)
    )
    lane = jax.lax.broadcasted_iota(jnp.int32, (1, _C), 1)
    neg_inf = jnp.float32(-jnp.inf)
    big = jnp.int32(_N)

    k0_ref[:, 0:_C] = ai_ref[...]
    k1_ref[:, 0:_C] = -ai_ref[...]
    k2_ref[:, 0:_C] = ao_ref[...]
    for _kref in (k0_ref, k1_ref, k2_ref):
        _kref[:, _C : 2 * _C] = p0_ref[...]
        _kref[:, 2 * _C : 3 * _C] = p1_ref[...]
    rm0_ref[...] = jnp.max(k0_ref[:, 0:_C].reshape(4, _C, _C), axis=2)
    rm1_ref[...] = jnp.max(k1_ref[:, 0:_C].reshape(4, _C, _C), axis=2)
    rm2_ref[...] = jnp.max(k2_ref[:, 0:_C].reshape(4, _C, _C), axis=2)

    def step(k_ref, rm_ref, orow):
        rm = rm_ref[...]
        mv = jnp.max(rm, keepdims=True)
        r0 = jnp.min(jnp.where(rm == mv, riota, big))
        row = k_ref[pl.ds(r0, 1), :]
        krow = row[:, 0:_C]
        sel0 = krow == mv
        c0v = jnp.min(jnp.where(sel0, lane, big), axis=1, keepdims=True)
        sel = lane == c0v
        nrow = jnp.where(sel, neg_inf, krow)
        k_ref[pl.ds(r0, 1), :] = jnp.concatenate([nrow, row[:, _C:]], axis=1)
        rm_ref[...] = jnp.where(riota == r0, jnp.max(nrow, keepdims=True), rm)
        p0 = jnp.sum(jnp.where(sel, row[:, _C : 2 * _C], 0.0), axis=1, keepdims=True)
        p1 = jnp.sum(
            jnp.where(sel, row[:, 2 * _C : 3 * _C], 0.0), axis=1, keepdims=True
        )
        lu_ref[pl.ds(orow, 1), :] = jnp.concatenate([p0, p1], axis=1)

    def body(j, _):
        step(k0_ref, rm0_ref, j)
        step(k1_ref, rm1_ref, j + _K)
        step(k2_ref, rm2_ref, j + 2 * _K)
        return 0

    jax.lax.fori_loop(0, _K, body, 0, unroll=5)

    lu = lu_ref[...]
    mx = jnp.max(lu, axis=1, keepdims=True)
    e = jnp.exp(lu - mx)
    ls_ref[...] = e / jnp.sum(e, axis=1, keepdims=True)


def kernel(bag_label, h, A, W, b):
    bl = jnp.asarray(bag_label)
    a0 = A[:, 0, 0]
    a1 = A[:, 0, 1]
    is_last = bl == (_NC - 1)
    ai = jnp.where(is_last, a1, a0).reshape(_R, _C)
    ao = jnp.where(is_last, a0, a1).reshape(_R, _C)

    b_pad = jnp.zeros((8, 128), jnp.float32).at[0, :_NC].set(b)

    p = pl.pallas_call(
        _logits_kernel,
        grid=(_N // _TILE,),
        in_specs=[
            pl.BlockSpec((_TILE, _DIM), lambda i: (i, 0)),
            pl.BlockSpec((_DIM, _NC), lambda i: (0, 0)),
            pl.BlockSpec((8, 128), lambda i: (0, 0)),
        ],
        out_specs=pl.BlockSpec((_TILE, _NC), lambda i: (i, 0)),
        out_shape=jax.ShapeDtypeStruct((_N, _NC), jnp.float32),
    )(h, W, b_pad)

    p0 = p[:, 0].reshape(_R, _C)
    p1 = p[:, 1].reshape(_R, _C)

    lu, ls = pl.pallas_call(
        _select_kernel,
        in_specs=[
            pl.BlockSpec((_R, _C), lambda: (0, 0)),
            pl.BlockSpec((_R, _C), lambda: (0, 0)),
            pl.BlockSpec((_R, _C), lambda: (0, 0)),
            pl.BlockSpec((_R, _C), lambda: (0, 0)),
        ],
        out_specs=[
            pl.BlockSpec((3 * _K, _NC), lambda: (0, 0)),
            pl.BlockSpec((3 * _K, _NC), lambda: (0, 0)),
        ],
        out_shape=[
            jax.ShapeDtypeStruct((3 * _K, _NC), jnp.float32),
            jax.ShapeDtypeStruct((3 * _K, _NC), jnp.float32),
        ],
        scratch_shapes=[
            pltpu.VMEM((_R, 3 * _C), jnp.float32),
            pltpu.VMEM((_R, 3 * _C), jnp.float32),
            pltpu.VMEM((_R, 3 * _C), jnp.float32),
            pltpu.VMEM((4, _C), jnp.float32),
            pltpu.VMEM((4, _C), jnp.float32),
            pltpu.VMEM((4, _C), jnp.float32),
        ],
    )(ai, ao, p0, p1)

    labels = jnp.concatenate(
        [
            jnp.ones((_K,), dtype=jnp.int32),
            jnp.zeros((2 * _K,), dtype=jnp.int32),
        ]
    )
    return labels, lu, ls
